# Initial kernel scaffold; baseline (speedup 1.0000x reference)
import functools

import jax
import jax.numpy as jnp
from jax import lax
from jax.experimental import pallas as pl
from jax.experimental.pallas import tpu as pltpu
from jax.experimental.pallas import tpu_sc as plsc

L = 16
ROW = 128
NC = 2
NS = 16


@functools.lru_cache(maxsize=None)
def _make_emb_kernel(V, D, B):
    NW = NC * NS
    assert B % (NW * ROW) == 0 and D == 2 * L
    per_w = B // NW
    n_idx_rows = per_w // ROW
    K = 5
    group = K * ROW
    n_group = per_w // group
    assert n_group % 2 == 0

    mesh = plsc.VectorSubcoreMesh(core_axis_name="c", subcore_axis_name="s")

    @functools.partial(
        pl.kernel,
        out_type=jax.ShapeDtypeStruct((B, D), jnp.float32),
        mesh=mesh,
        scratch_types=[
            pltpu.VMEM((n_idx_rows, ROW), jnp.int32),
            pltpu.VMEM((group, D), jnp.float32),
            pltpu.VMEM((group, D), jnp.float32),
            pltpu.VMEM((L,), jnp.float32),
            pltpu.SemaphoreType.DMA,
            pltpu.SemaphoreType.DMA,
        ],
    )
    def emb(table_hbm, idx_hbm, s_hbm, out_hbm, idx_v, r0, r1, sv, g0, g1):
        wid = lax.axis_index("s") * NC + lax.axis_index("c")
        base = wid * per_w
        pltpu.sync_copy(idx_hbm.at[wid], idx_v)
        pltpu.sync_copy(s_hbm, sv)
        s = jnp.exp(sv[...])

        rows = (r0, r1)
        gsem = (g0, g1)

        def fire(g, b):
            for j in range(K):
                pltpu.make_async_copy(
                    table_hbm.at[idx_v.at[g * K + j]],
                    rows[b].at[pl.ds(j * ROW, ROW)],
                    gsem[b],
                ).start()

        def drain(b):
            pltpu.make_async_copy(
                table_hbm.at[pl.ds(0, group)], rows[b], gsem[b]
            ).wait()

        def scale_rows(b):
            r = rows[b]

            def body(i, _):
                r[i, pl.ds(0, L)] = r[i, pl.ds(0, L)] * s
                r[i, pl.ds(L, L)] = r[i, pl.ds(L, L)] * s
                return 0

            lax.fori_loop(0, group, body, 0, unroll=4)

        def flush(g, b):
            pltpu.sync_copy(
                rows[b], out_hbm.at[pl.ds(base + g * group, group)]
            )

        def step(g, b):
            fire(g + 1, 1 - b)
            drain(b)
            scale_rows(b)
            flush(g, b)

        fire(0, 0)

        def outer(t, _):
            go = t * 2
            step(go, 0)
            step(go + 1, 1)
            return 0

        lax.fori_loop(0, (n_group - 2) // 2, outer, 0)
        fire(n_group - 1, 1)
        drain(0)
        scale_rows(0)
        flush(n_group - 2, 0)
        drain(1)
        scale_rows(1)
        flush(n_group - 1, 1)

    return emb, NW, n_idx_rows


def kernel(input, weight, scale):
    V, D = weight.shape
    B = input.size
    emb, NW, n_idx_rows = _make_emb_kernel(V, D, B)
    idx = input.reshape(NW, n_idx_rows, ROW).astype(jnp.int32)
    svec = jnp.full((L,), scale, dtype=jnp.float32)
    out = emb(weight, idx, svec)
    return out.reshape(input.shape + (D,))


# speedup vs baseline: 1.0477x; 1.0477x over previous
import functools

import jax
import jax.numpy as jnp
from jax import lax
from jax.experimental import pallas as pl
from jax.experimental.pallas import tpu as pltpu
from jax.experimental.pallas import tpu_sc as plsc

L = 16
ROW = 128
NC = 2
NS = 16


@functools.lru_cache(maxsize=None)
def _make_emb_kernel(V, D, B):
    NW = NC * NS
    assert B % (NW * ROW) == 0 and D == 2 * L
    per_w = B // NW
    n_idx_rows = per_w // ROW
    K = 5
    group = K * ROW
    n_group = per_w // group
    assert n_group % 2 == 0

    mesh = plsc.VectorSubcoreMesh(core_axis_name="c", subcore_axis_name="s")

    @functools.partial(
        pl.kernel,
        out_type=jax.ShapeDtypeStruct((B, D), jnp.float32),
        mesh=mesh,
        compiler_params=pltpu.CompilerParams(use_tc_tiling_on_sc=False),
        scratch_types=[
            pltpu.VMEM((n_idx_rows, ROW), jnp.int32),
            pltpu.VMEM((group, D), jnp.float32),
            pltpu.VMEM((group, D), jnp.float32),
            pltpu.VMEM((L,), jnp.float32),
            pltpu.SemaphoreType.DMA,
            pltpu.SemaphoreType.DMA,
        ],
    )
    def emb(table_hbm, idx_hbm, s_hbm, out_hbm, idx_v, r0, r1, sv, g0, g1):
        wid = lax.axis_index("s") * NC + lax.axis_index("c")
        base = wid * per_w
        pltpu.sync_copy(idx_hbm.at[wid], idx_v)
        pltpu.sync_copy(s_hbm, sv)
        s = jnp.exp(sv[...])

        rows = (r0, r1)
        gsem = (g0, g1)

        def fire(g, b):
            for j in range(K):
                pltpu.make_async_copy(
                    table_hbm.at[idx_v.at[g * K + j]],
                    rows[b].at[pl.ds(j * ROW, ROW)],
                    gsem[b],
                ).start()

        def drain(b):
            pltpu.make_async_copy(
                table_hbm.at[pl.ds(0, group)], rows[b], gsem[b]
            ).wait()

        def scale_rows(b):
            r = rows[b]

            def body(i, _):
                r[i, pl.ds(0, L)] = r[i, pl.ds(0, L)] * s
                r[i, pl.ds(L, L)] = r[i, pl.ds(L, L)] * s
                return 0

            lax.fori_loop(0, group, body, 0, unroll=4)

        def flush(g, b):
            pltpu.sync_copy(
                rows[b], out_hbm.at[pl.ds(base + g * group, group)]
            )

        def step(g, b):
            fire(g + 1, 1 - b)
            drain(b)
            scale_rows(b)
            flush(g, b)

        fire(0, 0)

        def outer(t, _):
            go = t * 2
            step(go, 0)
            step(go + 1, 1)
            return 0

        lax.fori_loop(0, (n_group - 2) // 2, outer, 0)
        fire(n_group - 1, 1)
        drain(0)
        scale_rows(0)
        flush(n_group - 2, 0)
        drain(1)
        scale_rows(1)
        flush(n_group - 1, 1)

    return emb, NW, n_idx_rows


def kernel(input, weight, scale):
    V, D = weight.shape
    B = input.size
    emb, NW, n_idx_rows = _make_emb_kernel(V, D, B)
    idx = input.reshape(NW, n_idx_rows, ROW).astype(jnp.int32)
    svec = jnp.full((L,), scale, dtype=jnp.float32)
    out = emb(weight, idx, svec)
    return out.reshape(input.shape + (D,))


# speedup vs baseline: 1.3484x; 1.2870x over previous
import functools

import jax
import jax.numpy as jnp
from jax import lax
from jax.experimental import pallas as pl
from jax.experimental.pallas import tpu as pltpu
from jax.experimental.pallas import tpu_sc as plsc

L = 16
ROW = 128
NC = 2
NS = 16


@functools.lru_cache(maxsize=None)
def _make_emb_kernel(V, D, NB, SEQ):
    NW = NC * NS
    iw = NB // NW
    K = iw // ROW
    group = K * ROW
    n_group = SEQ
    assert NB % (NW * ROW) == 0 and D == 2 * L and n_group % 2 == 0

    mesh = plsc.VectorSubcoreMesh(core_axis_name="c", subcore_axis_name="s")

    @functools.partial(
        pl.kernel,
        out_type=jax.ShapeDtypeStruct((SEQ, D, NB), jnp.float32),
        mesh=mesh,
        compiler_params=pltpu.CompilerParams(
            use_tc_tiling_on_sc=False, needs_layout_passes=False
        ),
        scratch_types=[
            pltpu.VMEM((SEQ * K, ROW), jnp.int32),
            pltpu.VMEM((group, D), jnp.float32),
            pltpu.VMEM((group, D), jnp.float32),
            pltpu.VMEM((D, group), jnp.float32),
            pltpu.VMEM((D, group), jnp.float32),
            pltpu.VMEM((L,), jnp.float32),
            pltpu.SemaphoreType.DMA,
            pltpu.SemaphoreType.DMA,
            pltpu.SemaphoreType.DMA,
            pltpu.SemaphoreType.DMA,
        ],
    )
    def emb(table_hbm, idx_hbm, s_hbm, out_hbm,
            idx_v, r0, r1, t0, t1, sv, g0, g1, o0, o1):
        wid = lax.axis_index("s") * NC + lax.axis_index("c")
        ibase = wid * iw
        pltpu.sync_copy(idx_hbm.at[wid], idx_v)
        pltpu.sync_copy(s_hbm, sv)
        s = jnp.exp(sv[...])
        riota = lax.iota(jnp.int32, L)

        rows = (r0, r1)
        tbuf = (t0, t1)
        gsem = (g0, g1)
        osem = (o0, o1)

        def fire(g, b):
            for k in range(K):
                pltpu.make_async_copy(
                    table_hbm.at[idx_v.at[g * K + k]],
                    rows[b].at[pl.ds(k * ROW, ROW)],
                    gsem[b],
                ).start()

        def drain(b):
            pltpu.make_async_copy(
                table_hbm.at[pl.ds(0, group)], rows[b], gsem[b]
            ).wait()

        def transpose_scale(b):
            r, t = rows[b], tbuf[b]

            def body_d(d, _):
                col = jnp.full((L,), d, dtype=jnp.int32)
                for m in range(group // L):
                    v = plsc.load_gather(r, [riota + m * L, col])
                    t[d, pl.ds(m * L, L)] = v * s
                return 0

            lax.fori_loop(0, D, body_d, 0)

        def flush(g, b):
            pltpu.make_async_copy(
                tbuf[b], out_hbm.at[g, :, pl.ds(ibase, group)], osem[b]
            ).start()

        def flush_wait(b):
            pltpu.make_async_copy(
                tbuf[b], out_hbm.at[0, :, pl.ds(ibase, group)], osem[b]
            ).wait()

        def step(g, b, *, prefetch=True, wait_out=True):
            if prefetch:
                fire(g + 1, 1 - b)
            drain(b)
            if wait_out:
                flush_wait(b)
            transpose_scale(b)
            flush(g, b)

        fire(0, 0)
        step(0, 0, wait_out=False)
        step(1, 1, wait_out=False)

        def outer(t, _):
            go = t * 2 + 2
            step(go, 0)
            step(go + 1, 1)
            return 0

        lax.fori_loop(0, (n_group - 4) // 2, outer, 0)
        step(n_group - 2, 0)
        step(n_group - 1, 1, prefetch=False)
        flush_wait(0)
        flush_wait(1)

    return emb, NW, K


def kernel(input, weight, scale):
    V, D = weight.shape
    NB, SEQ = input.shape
    emb, NW, K = _make_emb_kernel(V, D, NB, SEQ)
    idxT = input.T.astype(jnp.int32)
    idx = (idxT.reshape(SEQ, NW, K, ROW)
           .transpose(1, 0, 2, 3)
           .reshape(NW, SEQ * K, ROW))
    svec = jnp.full((L,), scale, dtype=jnp.float32)
    outT = emb(weight, idx, svec)
    return jnp.transpose(outT, (2, 0, 1))


# speedup vs baseline: 2.2798x; 1.6908x over previous
import functools

import jax
import jax.numpy as jnp
from jax import lax
from jax.experimental import pallas as pl
from jax.experimental.pallas import tpu as pltpu
from jax.experimental.pallas import tpu_sc as plsc

L = 16
ROW = 128
NC = 2
NS = 16


@functools.lru_cache(maxsize=None)
def _make_emb_kernel(V, D, NB, SEQ):
    NW = NC * NS
    iw = NB // NW
    K = iw // ROW
    group = K * ROW
    n_group = SEQ
    assert NB % (NW * ROW) == 0 and D == 2 * L and n_group % 2 == 0

    mesh = plsc.VectorSubcoreMesh(core_axis_name="c", subcore_axis_name="s")

    @functools.partial(
        pl.kernel,
        out_type=jax.ShapeDtypeStruct((SEQ, D, NB), jnp.float32),
        mesh=mesh,
        compiler_params=pltpu.CompilerParams(
            use_tc_tiling_on_sc=False, needs_layout_passes=False
        ),
        scratch_types=[
            pltpu.VMEM((SEQ * K, ROW), jnp.int32),
            pltpu.VMEM((group, D), jnp.float32),
            pltpu.VMEM((group, D), jnp.float32),
            pltpu.VMEM((D, group + 1), jnp.float32),
            pltpu.VMEM((D, group + 1), jnp.float32),
            pltpu.VMEM((L,), jnp.float32),
            pltpu.SemaphoreType.DMA,
            pltpu.SemaphoreType.DMA,
            pltpu.SemaphoreType.DMA,
            pltpu.SemaphoreType.DMA,
        ],
    )
    def emb(table_hbm, idx_hbm, s_hbm, out_hbm,
            idx_v, r0, r1, t0, t1, sv, g0, g1, o0, o1):
        wid = lax.axis_index("s") * NC + lax.axis_index("c")
        ibase = wid * iw
        pltpu.sync_copy(idx_hbm.at[wid], idx_v)
        pltpu.sync_copy(s_hbm, sv)
        s = jnp.exp(sv[...])
        riota = lax.iota(jnp.int32, L)

        rows = (r0, r1)
        tbuf = (t0, t1)
        gsem = (g0, g1)
        osem = (o0, o1)

        def fire(g, b):
            for k in range(K):
                pltpu.make_async_copy(
                    table_hbm.at[idx_v.at[g * K + k]],
                    rows[b].at[pl.ds(k * ROW, ROW)],
                    gsem[b],
                ).start()

        def drain(b):
            pltpu.make_async_copy(
                table_hbm.at[pl.ds(0, group)], rows[b], gsem[b]
            ).wait()

        def transpose_scale(b):
            r, t = rows[b], tbuf[b]
            hi = riota + L

            def body_i(i, _):
                col = jnp.full((L,), i, dtype=jnp.int32)
                v0 = r[i, pl.ds(0, L)]
                v1 = r[i, pl.ds(L, L)]
                plsc.store_scatter(t, [riota, col], v0 * s)
                plsc.store_scatter(t, [hi, col], v1 * s)
                return 0

            lax.fori_loop(0, group, body_i, 0, unroll=4)

        def flush(g, b):
            pltpu.make_async_copy(
                tbuf[b].at[:, pl.ds(0, group)],
                out_hbm.at[g, :, pl.ds(ibase, group)],
                osem[b],
            ).start()

        def flush_wait(b):
            pltpu.make_async_copy(
                tbuf[b].at[:, pl.ds(0, group)],
                out_hbm.at[0, :, pl.ds(ibase, group)],
                osem[b],
            ).wait()

        def step(g, b, *, prefetch=True, wait_out=True):
            if prefetch:
                fire(g + 1, 1 - b)
            drain(b)
            if wait_out:
                flush_wait(b)
            transpose_scale(b)
            flush(g, b)

        fire(0, 0)
        step(0, 0, wait_out=False)
        step(1, 1, wait_out=False)

        def outer(t, _):
            go = t * 2 + 2
            step(go, 0)
            step(go + 1, 1)
            return 0

        lax.fori_loop(0, (n_group - 4) // 2, outer, 0)
        step(n_group - 2, 0)
        step(n_group - 1, 1, prefetch=False)
        flush_wait(0)
        flush_wait(1)

    return emb, NW, K


def kernel(input, weight, scale):
    V, D = weight.shape
    NB, SEQ = input.shape
    emb, NW, K = _make_emb_kernel(V, D, NB, SEQ)
    idxT = input.T.astype(jnp.int32)
    idx = (idxT.reshape(SEQ, NW, K, ROW)
           .transpose(1, 0, 2, 3)
           .reshape(NW, SEQ * K, ROW))
    svec = jnp.full((L,), scale, dtype=jnp.float32)
    outT = emb(weight, idx, svec)
    return jnp.transpose(outT, (2, 0, 1))
